# baseline (speedup 1.0000x reference)
import functools

import jax
import jax.numpy as jnp
from jax import lax
from jax.experimental import pallas as pl
from jax.experimental.pallas import tpu as pltpu
from jax.experimental.pallas import tpu_sc as plsc

_T = 2048
_D = 1024
_E = 16
_H = 2048
_CAP = 256
_NSLOT = _E * _CAP
_NTRASH = 16
_NW = 32
_TPW = _T // _NW
_CH = 16


def _gating_body(x_ref, wg_ref, s0_ref, s1_ref, c0_ref, c1_ref,
                 g0_ref, g1_ref, xbf_ref):
    x = x_ref[...]
    xbf_ref[...] = x.astype(jnp.bfloat16)
    wg = wg_ref[...]
    logits = jnp.dot(x, wg, preferred_element_type=jnp.float32)
    m = jnp.max(logits, axis=-1, keepdims=True)
    ex = jnp.exp(logits - m)
    p = ex / jnp.sum(ex, axis=-1, keepdims=True)

    eio = lax.broadcasted_iota(jnp.int32, (_T, _E), 1)
    v0 = jnp.max(p, axis=-1, keepdims=True)
    i0 = jnp.min(jnp.where(p == v0, eio, _E), axis=-1, keepdims=True)
    m0 = (eio == i0).astype(jnp.float32)
    pm = jnp.where(eio == i0, -jnp.inf, p)
    v1 = jnp.max(pm, axis=-1, keepdims=True)
    i1 = jnp.min(jnp.where(pm == v1, eio, _E), axis=-1, keepdims=True)
    m1 = (eio == i1).astype(jnp.float32)

    c = jnp.concatenate([m0, m1], axis=-1)
    k = 1
    while k < _T:
        sh = jnp.concatenate(
            [jnp.zeros((k, 2 * _E), jnp.float32), c[:_T - k]], axis=0)
        c = c + sh
        k *= 2
    c0 = c[:, :_E]
    c1 = c[:, _E:]
    loc0 = c0 - m0
    loc1 = c1 - m1 + c0[_T - 1:_T, :]
    keep0 = jnp.where(loc0 < _CAP, 1.0, 0.0) * m0
    keep1 = jnp.where(loc1 < _CAP, 1.0, 0.0) * m1
    pos0 = jnp.sum(loc0 * keep0, axis=-1).astype(jnp.int32)
    pos1 = jnp.sum(loc1 * keep1, axis=-1).astype(jnp.int32)
    k0 = jnp.sum(keep0, axis=-1)
    k1 = jnp.sum(keep1, axis=-1)
    denom = v0[:, 0] + v1[:, 0] + 1e-9
    g0 = v0[:, 0] / denom * k0
    g1 = v1[:, 0] / denom * k1
    e0 = i0[:, 0]
    e1 = i1[:, 0]
    slot_c0 = e0 * _CAP + pos0
    slot_c1 = e1 * _CAP + pos1
    slot_d0 = jnp.where(k0 > 0.0, slot_c0, _NSLOT + e0)
    slot_d1 = jnp.where(k1 > 0.0, slot_c1, _NSLOT + e1)
    s0_ref[...] = slot_d0
    s1_ref[...] = slot_d1
    c0_ref[...] = slot_c0
    c1_ref[...] = slot_c1
    g0_ref[...] = jnp.broadcast_to(g0[:, None], (_T, 16))
    g1_ref[...] = jnp.broadcast_to(g1[:, None], (_T, 16))


_gating = pl.pallas_call(
    _gating_body,
    out_shape=[
        jax.ShapeDtypeStruct((_T,), jnp.int32),
        jax.ShapeDtypeStruct((_T,), jnp.int32),
        jax.ShapeDtypeStruct((_T,), jnp.int32),
        jax.ShapeDtypeStruct((_T,), jnp.int32),
        jax.ShapeDtypeStruct((_T, 16), jnp.float32),
        jax.ShapeDtypeStruct((_T, 16), jnp.float32),
        jax.ShapeDtypeStruct((_T, _D), jnp.bfloat16),
    ],
)


_DH = _TPW // 2


def _dispatch_body(x_hbm, s0_hbm, s1_hbm, xe_hbm, idx0_v, idx1_v, rows_v,
                   sem00, sem01, sem10, sem11):
    wid = lax.axis_index("s") * 2 + lax.axis_index("c")
    base = wid * _TPW
    sems = ((sem00, sem01), (sem10, sem11))
    pend = []
    for h in range(2):
        pltpu.sync_copy(s0_hbm.at[pl.ds(base + h * _DH, _DH)], idx0_v.at[h])
        pltpu.sync_copy(s1_hbm.at[pl.ds(base + h * _DH, _DH)], idx1_v.at[h])
        pltpu.sync_copy(x_hbm.at[pl.ds(base + h * _DH, _DH)], rows_v.at[h])
        pend.append(pltpu.async_copy(rows_v.at[h], xe_hbm.at[idx0_v.at[h]],
                                     sems[h][0]))
        pend.append(pltpu.async_copy(rows_v.at[h], xe_hbm.at[idx1_v.at[h]],
                                     sems[h][1]))
    for cp in pend:
        cp.wait()


@functools.cache
def _build_dispatch():
    mesh = plsc.VectorSubcoreMesh(core_axis_name="c", subcore_axis_name="s")
    return pl.kernel(
        _dispatch_body,
        out_type=jax.ShapeDtypeStruct((_NSLOT + _NTRASH, _D // 2), jnp.int32),
        mesh=mesh,
        scratch_types=[
            pltpu.VMEM((2, _DH), jnp.int32),
            pltpu.VMEM((2, _DH), jnp.int32),
            pltpu.VMEM((2, _DH, _D // 2), jnp.int32),
            pltpu.SemaphoreType.DMA,
            pltpu.SemaphoreType.DMA,
            pltpu.SemaphoreType.DMA,
            pltpu.SemaphoreType.DMA,
        ],
    )


def _ffn_body(xe_ref, w1_ref, b1_ref, w2_ref, b2_ref, yo_ref):
    h = jnp.maximum(
        jnp.dot(xe_ref[...].astype(jnp.float32), w1_ref[0],
                preferred_element_type=jnp.float32)
        + b1_ref[0], 0.0)
    yo_ref[...] = (
        jnp.dot(h, w2_ref[0], preferred_element_type=jnp.float32) + b2_ref[0])


_ffn = pl.pallas_call(
    _ffn_body,
    grid=(_E,),
    in_specs=[
        pl.BlockSpec((_CAP, _D), lambda e: (e, 0)),
        pl.BlockSpec((1, _D, _H), lambda e: (e, 0, 0)),
        pl.BlockSpec((1, 1, _H), lambda e: (e, 0, 0)),
        pl.BlockSpec((1, _H, _D), lambda e: (e, 0, 0)),
        pl.BlockSpec((1, 1, _D), lambda e: (e, 0, 0)),
    ],
    out_specs=pl.BlockSpec((_CAP, _D), lambda e: (e, 0)),
    out_shape=jax.ShapeDtypeStruct((_NSLOT, _D), jnp.float32),
)


_NCH = _TPW // _CH
_NBUF = 2


def _combine_body(yo_hbm, c0_hbm, c1_hbm, g0_hbm, g1_hbm, y_hbm,
                  idx0_v, idx1_v, g0_v, g1_v, r0_v, r1_v, out_v,
                  sem00, sem01, sem10, sem11):
    wid = lax.axis_index("s") * 2 + lax.axis_index("c")
    sems = ((sem00, sem01), (sem10, sem11))

    def fire(ch, b):
        base = wid * _TPW + ch * _CH
        pltpu.sync_copy(c0_hbm.at[pl.ds(base, _CH)], idx0_v.at[b])
        pltpu.sync_copy(c1_hbm.at[pl.ds(base, _CH)], idx1_v.at[b])
        pltpu.sync_copy(g0_hbm.at[pl.ds(base, _CH)], g0_v.at[b])
        pltpu.sync_copy(g1_hbm.at[pl.ds(base, _CH)], g1_v.at[b])
        return (pltpu.async_copy(yo_hbm.at[idx0_v.at[b]], r0_v.at[b], sems[b][0]),
                pltpu.async_copy(yo_hbm.at[idx1_v.at[b]], r1_v.at[b], sems[b][1]))

    pend = fire(0, 0)
    for ch in range(_NCH):
        b = ch % _NBUF
        nxt = None
        if ch + 1 < _NCH:
            nxt = fire(ch + 1, (ch + 1) % _NBUF)
        pend[0].wait()
        pend[1].wait()

        def _token(j, _):
            gv0 = g0_v[b, j, :]
            gv1 = g1_v[b, j, :]

            @plsc.parallel_loop(0, _D // 16, unroll=8)
            def _col(s):
                a = r0_v[b, j, pl.ds(s * 16, 16)]
                c = r1_v[b, j, pl.ds(s * 16, 16)]
                out_v[b, j, pl.ds(s * 16, 16)] = gv0 * a + gv1 * c

            return 0

        lax.fori_loop(0, _CH, _token, 0)
        pltpu.sync_copy(out_v.at[b], y_hbm.at[pl.ds(wid * _TPW + ch * _CH, _CH)])
        pend = nxt


@functools.cache
def _build_combine():
    mesh = plsc.VectorSubcoreMesh(core_axis_name="c", subcore_axis_name="s")
    return pl.kernel(
        _combine_body,
        out_type=jax.ShapeDtypeStruct((_T, _D), jnp.float32),
        mesh=mesh,
        scratch_types=[
            pltpu.VMEM((_NBUF, _CH), jnp.int32),
            pltpu.VMEM((_NBUF, _CH), jnp.int32),
            pltpu.VMEM((_NBUF, _CH, 16), jnp.float32),
            pltpu.VMEM((_NBUF, _CH, 16), jnp.float32),
            pltpu.VMEM((_NBUF, _CH, _D), jnp.float32),
            pltpu.VMEM((_NBUF, _CH, _D), jnp.float32),
            pltpu.VMEM((_NBUF, _CH, _D), jnp.float32),
            pltpu.SemaphoreType.DMA,
            pltpu.SemaphoreType.DMA,
            pltpu.SemaphoreType.DMA,
            pltpu.SemaphoreType.DMA,
        ],
    )


def kernel(input, wg, w1, b1, w2, b2):
    s0, s1, c0, c1, g0b, g1b, xbf = _gating(input, wg)
    x32 = lax.bitcast_convert_type(
        xbf.reshape(_T, _D // 2, 2), jnp.int32)
    xe32 = _build_dispatch()(x32, s0, s1)
    xe = lax.bitcast_convert_type(xe32, jnp.bfloat16).reshape(
        _NSLOT + _NTRASH, _D)
    yo = _ffn(xe, w1, b1.reshape(_E, 1, _H), w2, b2.reshape(_E, 1, _D))
    y = _build_combine()(yo, c0, c1, g0b, g1b)
    return y


# speedup vs baseline: 1.9955x; 1.9955x over previous
import functools

import jax
import jax.numpy as jnp
from jax import lax
from jax.experimental import pallas as pl
from jax.experimental.pallas import tpu as pltpu
from jax.experimental.pallas import tpu_sc as plsc

_T = 2048
_D = 1024
_E = 16
_H = 2048
_CAP = 256
_NSLOT = _E * _CAP
_NTRASH = 16
_NW = 32
_TPW = _T // _NW
_CH = 16


def _gating_body(x_ref, wg_ref, s0_ref, s1_ref, c0_ref, c1_ref,
                 g0_ref, g1_ref, xbf_ref):
    x = x_ref[...]
    xu = lax.bitcast_convert_type(x.astype(jnp.bfloat16), jnp.uint16)
    lo = xu[:, :_D // 2].astype(jnp.uint32)
    hi = xu[:, _D // 2:].astype(jnp.uint32)
    xbf_ref[...] = lax.bitcast_convert_type(lo | (hi << 16), jnp.int32)
    wg = wg_ref[...]
    logits = jnp.dot(x, wg, preferred_element_type=jnp.float32)
    m = jnp.max(logits, axis=-1, keepdims=True)
    ex = jnp.exp(logits - m)
    p = ex / jnp.sum(ex, axis=-1, keepdims=True)

    eio = lax.broadcasted_iota(jnp.int32, (_T, _E), 1)
    v0 = jnp.max(p, axis=-1, keepdims=True)
    i0 = jnp.min(jnp.where(p == v0, eio, _E), axis=-1, keepdims=True)
    m0 = (eio == i0).astype(jnp.float32)
    pm = jnp.where(eio == i0, -jnp.inf, p)
    v1 = jnp.max(pm, axis=-1, keepdims=True)
    i1 = jnp.min(jnp.where(pm == v1, eio, _E), axis=-1, keepdims=True)
    m1 = (eio == i1).astype(jnp.float32)

    c = jnp.concatenate([m0, m1], axis=-1)
    k = 1
    while k < _T:
        sh = jnp.concatenate(
            [jnp.zeros((k, 2 * _E), jnp.float32), c[:_T - k]], axis=0)
        c = c + sh
        k *= 2
    c0 = c[:, :_E]
    c1 = c[:, _E:]
    loc0 = c0 - m0
    loc1 = c1 - m1 + c0[_T - 1:_T, :]
    keep0 = jnp.where(loc0 < _CAP, 1.0, 0.0) * m0
    keep1 = jnp.where(loc1 < _CAP, 1.0, 0.0) * m1
    pos0 = jnp.sum(loc0 * keep0, axis=-1).astype(jnp.int32)
    pos1 = jnp.sum(loc1 * keep1, axis=-1).astype(jnp.int32)
    k0 = jnp.sum(keep0, axis=-1)
    k1 = jnp.sum(keep1, axis=-1)
    denom = v0[:, 0] + v1[:, 0] + 1e-9
    g0 = v0[:, 0] / denom * k0
    g1 = v1[:, 0] / denom * k1
    e0 = i0[:, 0]
    e1 = i1[:, 0]
    slot_c0 = e0 * _CAP + pos0
    slot_c1 = e1 * _CAP + pos1
    slot_d0 = jnp.where(k0 > 0.0, slot_c0, _NSLOT + e0)
    slot_d1 = jnp.where(k1 > 0.0, slot_c1, _NSLOT + e1)
    s0_ref[...] = slot_d0
    s1_ref[...] = slot_d1
    c0_ref[...] = slot_c0
    c1_ref[...] = slot_c1
    g0_ref[...] = jnp.broadcast_to(g0[:, None], (_T, 16))
    g1_ref[...] = jnp.broadcast_to(g1[:, None], (_T, 16))


_gating = pl.pallas_call(
    _gating_body,
    out_shape=[
        jax.ShapeDtypeStruct((_T,), jnp.int32),
        jax.ShapeDtypeStruct((_T,), jnp.int32),
        jax.ShapeDtypeStruct((_T,), jnp.int32),
        jax.ShapeDtypeStruct((_T,), jnp.int32),
        jax.ShapeDtypeStruct((_T, 16), jnp.float32),
        jax.ShapeDtypeStruct((_T, 16), jnp.float32),
        jax.ShapeDtypeStruct((_T, _D // 2), jnp.int32),
    ],
)


_DH = _TPW // 2


def _dispatch_body(x_hbm, s0_hbm, s1_hbm, xe_hbm, idx0_v, idx1_v, rows_v,
                   sem00, sem01, sem10, sem11):
    wid = lax.axis_index("s") * 2 + lax.axis_index("c")
    base = wid * _TPW
    sems = ((sem00, sem01), (sem10, sem11))
    pend = []
    for h in range(2):
        pltpu.sync_copy(s0_hbm.at[pl.ds(base + h * _DH, _DH)], idx0_v.at[h])
        pltpu.sync_copy(s1_hbm.at[pl.ds(base + h * _DH, _DH)], idx1_v.at[h])
        pltpu.sync_copy(x_hbm.at[pl.ds(base + h * _DH, _DH)], rows_v.at[h])
        pend.append(pltpu.async_copy(rows_v.at[h], xe_hbm.at[idx0_v.at[h]],
                                     sems[h][0]))
        pend.append(pltpu.async_copy(rows_v.at[h], xe_hbm.at[idx1_v.at[h]],
                                     sems[h][1]))
    for cp in pend:
        cp.wait()


@functools.cache
def _build_dispatch():
    mesh = plsc.VectorSubcoreMesh(core_axis_name="c", subcore_axis_name="s")
    return pl.kernel(
        _dispatch_body,
        out_type=jax.ShapeDtypeStruct((_NSLOT + _NTRASH, _D // 2), jnp.int32),
        mesh=mesh,
        scratch_types=[
            pltpu.VMEM((2, _DH), jnp.int32),
            pltpu.VMEM((2, _DH), jnp.int32),
            pltpu.VMEM((2, _DH, _D // 2), jnp.int32),
            pltpu.SemaphoreType.DMA,
            pltpu.SemaphoreType.DMA,
            pltpu.SemaphoreType.DMA,
            pltpu.SemaphoreType.DMA,
        ],
    )


def _ffn_body(xe_ref, w1_ref, b1_ref, w2_ref, b2_ref, yo_ref):
    xu = lax.bitcast_convert_type(xe_ref[...], jnp.uint32)
    lo = lax.bitcast_convert_type(
        (xu & 0xFFFF).astype(jnp.uint16), jnp.bfloat16)
    hi = lax.bitcast_convert_type(
        (xu >> 16).astype(jnp.uint16), jnp.bfloat16)
    xb = jnp.concatenate([lo, hi], axis=1).astype(jnp.float32)
    h = jnp.maximum(
        jnp.dot(xb, w1_ref[0], preferred_element_type=jnp.float32)
        + b1_ref[0], 0.0)
    yo_ref[...] = (
        jnp.dot(h, w2_ref[0], preferred_element_type=jnp.float32) + b2_ref[0])


_ffn = pl.pallas_call(
    _ffn_body,
    grid=(_E,),
    in_specs=[
        pl.BlockSpec((_CAP, _D // 2), lambda e: (e, 0)),
        pl.BlockSpec((1, _D, _H), lambda e: (e, 0, 0)),
        pl.BlockSpec((1, 1, _H), lambda e: (e, 0, 0)),
        pl.BlockSpec((1, _H, _D), lambda e: (e, 0, 0)),
        pl.BlockSpec((1, 1, _D), lambda e: (e, 0, 0)),
    ],
    out_specs=pl.BlockSpec((_CAP, _D), lambda e: (e, 0)),
    out_shape=jax.ShapeDtypeStruct((_NSLOT, _D), jnp.float32),
)


_NCH = _TPW // _CH
_NBUF = 2


def _combine_body(yo_hbm, c0_hbm, c1_hbm, g0_hbm, g1_hbm, y_hbm,
                  idx0_v, idx1_v, g0_v, g1_v, r0_v, r1_v, out_v,
                  sem00, sem01, sem10, sem11):
    wid = lax.axis_index("s") * 2 + lax.axis_index("c")
    sems = ((sem00, sem01), (sem10, sem11))

    def fire(ch, b):
        base = wid * _TPW + ch * _CH
        pltpu.sync_copy(c0_hbm.at[pl.ds(base, _CH)], idx0_v.at[b])
        pltpu.sync_copy(c1_hbm.at[pl.ds(base, _CH)], idx1_v.at[b])
        pltpu.sync_copy(g0_hbm.at[pl.ds(base, _CH)], g0_v.at[b])
        pltpu.sync_copy(g1_hbm.at[pl.ds(base, _CH)], g1_v.at[b])
        return (pltpu.async_copy(yo_hbm.at[idx0_v.at[b]], r0_v.at[b], sems[b][0]),
                pltpu.async_copy(yo_hbm.at[idx1_v.at[b]], r1_v.at[b], sems[b][1]))

    pend = fire(0, 0)
    for ch in range(_NCH):
        b = ch % _NBUF
        nxt = None
        if ch + 1 < _NCH:
            nxt = fire(ch + 1, (ch + 1) % _NBUF)
        pend[0].wait()
        pend[1].wait()

        def _token(j, _):
            gv0 = g0_v[b, j, :]
            gv1 = g1_v[b, j, :]

            @plsc.parallel_loop(0, _D // 16, unroll=8)
            def _col(s):
                a = r0_v[b, j, pl.ds(s * 16, 16)]
                c = r1_v[b, j, pl.ds(s * 16, 16)]
                out_v[b, j, pl.ds(s * 16, 16)] = gv0 * a + gv1 * c

            return 0

        lax.fori_loop(0, _CH, _token, 0)
        pltpu.sync_copy(out_v.at[b], y_hbm.at[pl.ds(wid * _TPW + ch * _CH, _CH)])
        pend = nxt


@functools.cache
def _build_combine():
    mesh = plsc.VectorSubcoreMesh(core_axis_name="c", subcore_axis_name="s")
    return pl.kernel(
        _combine_body,
        out_type=jax.ShapeDtypeStruct((_T, _D), jnp.float32),
        mesh=mesh,
        scratch_types=[
            pltpu.VMEM((_NBUF, _CH), jnp.int32),
            pltpu.VMEM((_NBUF, _CH), jnp.int32),
            pltpu.VMEM((_NBUF, _CH, 16), jnp.float32),
            pltpu.VMEM((_NBUF, _CH, 16), jnp.float32),
            pltpu.VMEM((_NBUF, _CH, _D), jnp.float32),
            pltpu.VMEM((_NBUF, _CH, _D), jnp.float32),
            pltpu.VMEM((_NBUF, _CH, _D), jnp.float32),
            pltpu.SemaphoreType.DMA,
            pltpu.SemaphoreType.DMA,
            pltpu.SemaphoreType.DMA,
            pltpu.SemaphoreType.DMA,
        ],
    )


def kernel(input, wg, w1, b1, w2, b2):
    s0, s1, c0, c1, g0b, g1b, x32 = _gating(input, wg)
    xe32 = _build_dispatch()(x32, s0, s1)
    yo = _ffn(xe32, w1, b1.reshape(_E, 1, _H), w2, b2.reshape(_E, 1, _D))
    y = _build_combine()(yo, c0, c1, g0b, g1b)
    return y


# speedup vs baseline: 2.0223x; 1.0134x over previous
import functools

import jax
import jax.numpy as jnp
from jax import lax
from jax.experimental import pallas as pl
from jax.experimental.pallas import tpu as pltpu
from jax.experimental.pallas import tpu_sc as plsc

_T = 2048
_D = 1024
_E = 16
_H = 2048
_CAP = 256
_NSLOT = _E * _CAP
_NTRASH = 16
_NW = 32
_TPW = _T // _NW
_CH = 16


def _gating_body(x_ref, wg_ref, s0_ref, s1_ref, c0_ref, c1_ref,
                 g0_ref, g1_ref, xbf_ref):
    x = x_ref[...]
    xu = lax.bitcast_convert_type(x.astype(jnp.bfloat16), jnp.uint16)
    lo = xu[:, :_D // 2].astype(jnp.uint32)
    hi = xu[:, _D // 2:].astype(jnp.uint32)
    xbf_ref[...] = lax.bitcast_convert_type(lo | (hi << 16), jnp.int32)
    wg = wg_ref[...]
    logits = jnp.dot(x, wg, preferred_element_type=jnp.float32)
    m = jnp.max(logits, axis=-1, keepdims=True)
    ex = jnp.exp(logits - m)
    p = ex / jnp.sum(ex, axis=-1, keepdims=True)

    eio = lax.broadcasted_iota(jnp.int32, (_T, _E), 1)
    v0 = jnp.max(p, axis=-1, keepdims=True)
    i0 = jnp.min(jnp.where(p == v0, eio, _E), axis=-1, keepdims=True)
    m0 = (eio == i0).astype(jnp.float32)
    pm = jnp.where(eio == i0, -jnp.inf, p)
    v1 = jnp.max(pm, axis=-1, keepdims=True)
    i1 = jnp.min(jnp.where(pm == v1, eio, _E), axis=-1, keepdims=True)
    m1 = (eio == i1).astype(jnp.float32)

    c = jnp.concatenate([m0, m1], axis=-1)
    k = 1
    while k < _T:
        sh = jnp.concatenate(
            [jnp.zeros((k, 2 * _E), jnp.float32), c[:_T - k]], axis=0)
        c = c + sh
        k *= 2
    c0 = c[:, :_E]
    c1 = c[:, _E:]
    loc0 = c0 - m0
    loc1 = c1 - m1 + c0[_T - 1:_T, :]
    keep0 = jnp.where(loc0 < _CAP, 1.0, 0.0) * m0
    keep1 = jnp.where(loc1 < _CAP, 1.0, 0.0) * m1
    pos0 = jnp.sum(loc0 * keep0, axis=-1).astype(jnp.int32)
    pos1 = jnp.sum(loc1 * keep1, axis=-1).astype(jnp.int32)
    k0 = jnp.sum(keep0, axis=-1)
    k1 = jnp.sum(keep1, axis=-1)
    denom = v0[:, 0] + v1[:, 0] + 1e-9
    g0 = v0[:, 0] / denom * k0
    g1 = v1[:, 0] / denom * k1
    e0 = i0[:, 0]
    e1 = i1[:, 0]
    slot_c0 = e0 * _CAP + pos0
    slot_c1 = e1 * _CAP + pos1
    slot_d0 = jnp.where(k0 > 0.0, slot_c0, _NSLOT + e0)
    slot_d1 = jnp.where(k1 > 0.0, slot_c1, _NSLOT + e1)
    s0_ref[...] = slot_d0
    s1_ref[...] = slot_d1
    c0_ref[...] = slot_c0
    c1_ref[...] = slot_c1
    g0_ref[...] = jnp.broadcast_to(g0[:, None], (_T, 16))
    g1_ref[...] = jnp.broadcast_to(g1[:, None], (_T, 16))


_gating = pl.pallas_call(
    _gating_body,
    out_shape=[
        jax.ShapeDtypeStruct((_T,), jnp.int32),
        jax.ShapeDtypeStruct((_T,), jnp.int32),
        jax.ShapeDtypeStruct((_T,), jnp.int32),
        jax.ShapeDtypeStruct((_T,), jnp.int32),
        jax.ShapeDtypeStruct((_T, 16), jnp.float32),
        jax.ShapeDtypeStruct((_T, 16), jnp.float32),
        jax.ShapeDtypeStruct((_T, _D // 2), jnp.int32),
    ],
)


_DH = _TPW // 2


def _dispatch_body(x_hbm, s0_hbm, s1_hbm, xe_hbm, idx0_v, idx1_v, rows_v,
                   sem00, sem01, sem10, sem11):
    wid = lax.axis_index("s") * 2 + lax.axis_index("c")
    base = wid * _TPW
    sems = ((sem00, sem01), (sem10, sem11))
    pend = []
    for h in range(2):
        pltpu.sync_copy(s0_hbm.at[pl.ds(base + h * _DH, _DH)], idx0_v.at[h])
        pltpu.sync_copy(s1_hbm.at[pl.ds(base + h * _DH, _DH)], idx1_v.at[h])
        pltpu.sync_copy(x_hbm.at[pl.ds(base + h * _DH, _DH)], rows_v.at[h])
        pend.append(pltpu.async_copy(rows_v.at[h], xe_hbm.at[idx0_v.at[h]],
                                     sems[h][0]))
        pend.append(pltpu.async_copy(rows_v.at[h], xe_hbm.at[idx1_v.at[h]],
                                     sems[h][1]))
    for cp in pend:
        cp.wait()


@functools.cache
def _build_dispatch():
    mesh = plsc.VectorSubcoreMesh(core_axis_name="c", subcore_axis_name="s")
    return pl.kernel(
        _dispatch_body,
        out_type=jax.ShapeDtypeStruct((_NSLOT + _NTRASH, _D // 2), jnp.int32),
        mesh=mesh,
        scratch_types=[
            pltpu.VMEM((2, _DH), jnp.int32),
            pltpu.VMEM((2, _DH), jnp.int32),
            pltpu.VMEM((2, _DH, _D // 2), jnp.int32),
            pltpu.SemaphoreType.DMA,
            pltpu.SemaphoreType.DMA,
            pltpu.SemaphoreType.DMA,
            pltpu.SemaphoreType.DMA,
        ],
    )


def _ffn_body(xe_ref, w1_ref, b1_ref, w2_ref, b2_ref, yo_ref):
    xu = lax.bitcast_convert_type(xe_ref[...], jnp.uint32)
    lo = lax.bitcast_convert_type(
        (xu & 0xFFFF).astype(jnp.uint16), jnp.bfloat16)
    hi = lax.bitcast_convert_type(
        (xu >> 16).astype(jnp.uint16), jnp.bfloat16)
    xb = jnp.concatenate([lo, hi], axis=1).astype(jnp.float32)
    h = jnp.maximum(
        jnp.dot(xb, w1_ref[0], preferred_element_type=jnp.float32)
        + b1_ref[0], 0.0)
    yo = jnp.dot(h, w2_ref[0], preferred_element_type=jnp.float32) + b2_ref[0]
    yb = lax.bitcast_convert_type(yo.astype(jnp.bfloat16), jnp.uint16)
    ylo = yb[:, :_D // 2].astype(jnp.uint32)
    yhi = yb[:, _D // 2:].astype(jnp.uint32)
    yo_ref[...] = lax.bitcast_convert_type(ylo | (yhi << 16), jnp.int32)


_ffn = pl.pallas_call(
    _ffn_body,
    grid=(_E,),
    in_specs=[
        pl.BlockSpec((_CAP, _D // 2), lambda e: (e, 0)),
        pl.BlockSpec((1, _D, _H), lambda e: (e, 0, 0)),
        pl.BlockSpec((1, 1, _H), lambda e: (e, 0, 0)),
        pl.BlockSpec((1, _H, _D), lambda e: (e, 0, 0)),
        pl.BlockSpec((1, 1, _D), lambda e: (e, 0, 0)),
    ],
    out_specs=pl.BlockSpec((_CAP, _D // 2), lambda e: (e, 0)),
    out_shape=jax.ShapeDtypeStruct((_NSLOT, _D // 2), jnp.int32),
)


_NCH = _TPW // _CH
_NBUF = 2


def _combine_body(yo_hbm, c0_hbm, c1_hbm, g0_hbm, g1_hbm, y_hbm,
                  idx0_v, idx1_v, g0_v, g1_v, r0_v, r1_v, out_v,
                  sem00, sem01, sem10, sem11):
    wid = lax.axis_index("s") * 2 + lax.axis_index("c")
    sems = ((sem00, sem01), (sem10, sem11))

    def fire(ch, b):
        base = wid * _TPW + ch * _CH
        pltpu.sync_copy(c0_hbm.at[pl.ds(base, _CH)], idx0_v.at[b])
        pltpu.sync_copy(c1_hbm.at[pl.ds(base, _CH)], idx1_v.at[b])
        pltpu.sync_copy(g0_hbm.at[pl.ds(base, _CH)], g0_v.at[b])
        pltpu.sync_copy(g1_hbm.at[pl.ds(base, _CH)], g1_v.at[b])
        return (pltpu.async_copy(yo_hbm.at[idx0_v.at[b]], r0_v.at[b], sems[b][0]),
                pltpu.async_copy(yo_hbm.at[idx1_v.at[b]], r1_v.at[b], sems[b][1]))

    pend = fire(0, 0)
    for ch in range(_NCH):
        b = ch % _NBUF
        nxt = None
        if ch + 1 < _NCH:
            nxt = fire(ch + 1, (ch + 1) % _NBUF)
        pend[0].wait()
        pend[1].wait()

        def _token(j, _):
            gv0 = g0_v[b, j, :]
            gv1 = g1_v[b, j, :]

            @plsc.parallel_loop(0, _D // 32, unroll=4)
            def _col(s):
                w0 = r0_v[b, j, pl.ds(s * 16, 16)]
                w1 = r1_v[b, j, pl.ds(s * 16, 16)]
                lo0 = lax.bitcast_convert_type(w0 << 16, jnp.float32)
                lo1 = lax.bitcast_convert_type(w1 << 16, jnp.float32)
                hi0 = lax.bitcast_convert_type(w0 & (-65536), jnp.float32)
                hi1 = lax.bitcast_convert_type(w1 & (-65536), jnp.float32)
                out_v[b, j, pl.ds(s * 16, 16)] = gv0 * lo0 + gv1 * lo1
                out_v[b, j, pl.ds(_D // 2 + s * 16, 16)] = (
                    gv0 * hi0 + gv1 * hi1)

            return 0

        lax.fori_loop(0, _CH, _token, 0)
        pltpu.sync_copy(out_v.at[b], y_hbm.at[pl.ds(wid * _TPW + ch * _CH, _CH)])
        pend = nxt


@functools.cache
def _build_combine():
    mesh = plsc.VectorSubcoreMesh(core_axis_name="c", subcore_axis_name="s")
    return pl.kernel(
        _combine_body,
        out_type=jax.ShapeDtypeStruct((_T, _D), jnp.float32),
        mesh=mesh,
        scratch_types=[
            pltpu.VMEM((_NBUF, _CH), jnp.int32),
            pltpu.VMEM((_NBUF, _CH), jnp.int32),
            pltpu.VMEM((_NBUF, _CH, 16), jnp.float32),
            pltpu.VMEM((_NBUF, _CH, 16), jnp.float32),
            pltpu.VMEM((_NBUF, _CH, _D // 2), jnp.int32),
            pltpu.VMEM((_NBUF, _CH, _D // 2), jnp.int32),
            pltpu.VMEM((_NBUF, _CH, _D), jnp.float32),
            pltpu.SemaphoreType.DMA,
            pltpu.SemaphoreType.DMA,
            pltpu.SemaphoreType.DMA,
            pltpu.SemaphoreType.DMA,
        ],
    )


def kernel(input, wg, w1, b1, w2, b2):
    s0, s1, c0, c1, g0b, g1b, x32 = _gating(input, wg)
    xe32 = _build_dispatch()(x32, s0, s1)
    yo = _ffn(xe32, w1, b1.reshape(_E, 1, _H), w2, b2.reshape(_E, 1, _D))
    y = _build_combine()(yo, c0, c1, g0b, g1b)
    return y
